# baseline (speedup 1.0000x reference)
import jax
import jax.numpy as jnp
from jax import lax
from jax.experimental import pallas as pl
from jax.experimental.pallas import tpu as pltpu
from jax.experimental.pallas import tpu_sc as plsc

N = 50000
E = 800000
H = 32
NC = 4
EC = 4

SC_CORES = 2
SC_SUBCORES = 16
SC_WORKERS = SC_CORES * SC_SUBCORES
LANES = 16

K_EDGES = 200
SUB = 100
NSUB = K_EDGES // SUB
NCHUNKS = E // K_EDGES
T_STEPS = -(-NCHUNKS // SC_WORKERS)
ROWS_PER_TILE = 3128
ROWS_LAST = N - (SC_SUBCORES - 1) * ROWS_PER_TILE

NB = 4
NRB = N // 4 // NB


def _dot(x, w):
    return jnp.dot(x.astype(jnp.bfloat16), w.astype(jnp.bfloat16),
                   preferred_element_type=jnp.float32)


def _kron(w, p):
    return jnp.kron(jnp.eye(p, dtype=jnp.float32), w)


def _tileb(b, p):
    return jnp.tile(b.reshape(-1), p).reshape(1, -1)


def _prep_node_body(nl4, WpK, bpK, WsK, WdK, bcK, nf4, a4, d4):
    nf = _dot(nl4[0], WpK[...]) + bpK[...]
    nf4[0] = nf
    a4[0] = _dot(nf, WsK[...])
    d4[0] = _dot(nf, WdK[...]) + bcK[...]


def _prep_node(nl4, WpK, bpK, WsK, WdK, bcK):
    full = lambda s: pl.BlockSpec(s, lambda i: (0,) * len(s))
    blk = lambda m: pl.BlockSpec((1, NRB, m), lambda i: (i, 0, 0))
    o = jax.ShapeDtypeStruct((NB, NRB, 128), jnp.float32)
    return pl.pallas_call(
        _prep_node_body,
        grid=(NB,),
        out_shape=(o, o, o),
        in_specs=[blk(16), full((16, 128)), full((1, 128)),
                  full((128, 128)), full((128, 128)), full((1, 128))],
        out_specs=(blk(128), blk(128), blk(128)),
    )(nl4.reshape(NB, NRB, 16), WpK, bpK, WsK, WdK, bcK)


def _prep_edge_body(el16, W1K, b1K, W2K, b2K, ef4):
    t = jnp.maximum(_dot(el16[...], W1K[...]) + b1K[...], 0.0)
    e = _dot(t, W2K[...]) + b2K[...]
    ef4[...] = e.reshape(ef4.shape)


def _prep_edge(el16, W1K, b1K, W2K, b2K):
    RE = 2000
    g = (E // 16) // RE
    full = lambda s: pl.BlockSpec(s, lambda i: (0,) * len(s))
    return pl.pallas_call(
        _prep_edge_body,
        grid=(g,),
        out_shape=jax.ShapeDtypeStruct((E // 4, 128), jnp.float32),
        in_specs=[pl.BlockSpec((RE, 64), lambda i: (i, 0)),
                  full((64, 512)), full((1, 512)), full((512, 512)),
                  full((1, 512))],
        out_specs=pl.BlockSpec((4 * RE, 128), lambda i: (i, 0)),
    )(el16, W1K, b1K, W2K, b2K)


def _msg_body(comb4, WvK, bvK, WoK, boK, msg4):
    t = _dot(comb4[...], WvK[...]) + bvK[...]
    msg4[...] = _dot(t, WoK[...]) + boK[...]


def _msg(comb4, WvK, bvK, WoK, boK):
    RE = 8000
    g = (E // 4) // RE
    full = lambda s: pl.BlockSpec(s, lambda i: (0,) * len(s))
    return pl.pallas_call(
        _msg_body,
        grid=(g,),
        out_shape=jax.ShapeDtypeStruct((E // 4, 128), jnp.float32),
        in_specs=[pl.BlockSpec((RE, 128), lambda i: (i, 0)),
                  full((128, 128)), full((1, 128)), full((128, 128)),
                  full((1, 128))],
        out_specs=pl.BlockSpec((RE, 128), lambda i: (i, 0)),
    )(comb4, WvK, bvK, WoK, boK)


def _node_update_body(nf4, agg0, agg1, Wc1K, bc1K, Wc2K, bc2K, WsK, WdK, bcK,
                      nf4o, a4, d4):
    h = nf4[0] + agg0[0] + agg1[0]
    t = jnp.maximum(_dot(h, Wc1K[...]) + bc1K[...], 0.0)
    nf2 = jnp.maximum(_dot(t, Wc2K[...]) + bc2K[...], 0.0)
    nf4o[0] = nf2
    a4[0] = _dot(nf2, WsK[...])
    d4[0] = _dot(nf2, WdK[...]) + bcK[...]


def _node_update(nf4, agg4, Wc1K, bc1K, Wc2K, bc2K, WsK, WdK, bcK):
    full = lambda s: pl.BlockSpec(s, lambda i: (0,) * len(s))
    blk = lambda: pl.BlockSpec((1, NRB, 128), lambda i: (i, 0, 0))
    o = jax.ShapeDtypeStruct((NB, NRB, 128), jnp.float32)
    r3 = lambda x: x.reshape(NB, NRB, 128)
    return pl.pallas_call(
        _node_update_body,
        grid=(NB,),
        out_shape=(o, o, o),
        in_specs=[blk(), blk(), blk(),
                  full((128, 128)), full((1, 128)), full((128, 128)),
                  full((1, 128)), full((128, 128)), full((128, 128)),
                  full((1, 128))],
        out_specs=(blk(), blk(), blk()),
    )(r3(nf4), r3(agg4[0]), r3(agg4[1]), Wc1K, bc1K, Wc2K, bc2K, WsK, WdK,
      bcK)


def _node_final_body(nf4, agg0, agg1, Wc1K, bc1K, Wc2K, bc2K, WnhK, bnhK,
                     out):
    h = nf4[0] + agg0[0] + agg1[0]
    t = jnp.maximum(_dot(h, Wc1K[...]) + bc1K[...], 0.0)
    nf2 = jnp.maximum(_dot(t, Wc2K[...]) + bc2K[...], 0.0)
    out[0] = _dot(nf2, WnhK[...]) + bnhK[...]


def _node_final(nf4, agg4, Wc1K, bc1K, Wc2K, bc2K, WnhK, bnhK):
    full = lambda s: pl.BlockSpec(s, lambda i: (0,) * len(s))
    blk = lambda m: pl.BlockSpec((1, NRB, m), lambda i: (i, 0, 0))
    r3 = lambda x: x.reshape(NB, NRB, 128)
    return pl.pallas_call(
        _node_final_body,
        grid=(NB,),
        out_shape=jax.ShapeDtypeStruct((NB, NRB, 16), jnp.float32),
        in_specs=[blk(128), blk(128), blk(128),
                  full((128, 128)), full((1, 128)), full((128, 128)),
                  full((1, 128)), full((128, 16)), full((1, 16))],
        out_specs=blk(16),
    )(r3(nf4), r3(agg4[0]), r3(agg4[1]), Wc1K, bc1K, Wc2K, bc2K, WnhK, bnhK)


def _edge_head_body(ef4, WehP, behP, out):
    x3 = ef4[...].reshape(ef4.shape[0] // 4, 4, 128)
    acc = behP[...]
    for m in range(4):
        acc = acc + _dot(x3[:, m, :], WehP[m])
    out[...] = acc


def _edge_head(ef4, WehP, behP):
    RE = 8000
    g = (E // 4) // RE
    full = lambda s: pl.BlockSpec(s, lambda i: (0,) * len(s))
    return pl.pallas_call(
        _edge_head_body,
        grid=(g,),
        out_shape=jax.ShapeDtypeStruct((E // 16, 64), jnp.float32),
        in_specs=[pl.BlockSpec((RE, 128), lambda i: (i, 0)),
                  full((4, 128, 64)), full((1, 64))],
        out_specs=pl.BlockSpec((RE // 4, 64), lambda i: (i, 0)),
    )(ef4, WehP, behP)


def _sc_comb_body(A_hbm, B_hbm, src_hbm, dst_hbm, comb_hbm,
                  sidx, didx, ab, bb, sem):
    c = lax.axis_index("c")
    s = lax.axis_index("s")
    w = s * SC_CORES + c

    def _chunk(t, _):
        chunk = w + t * SC_WORKERS

        @pl.when(chunk < NCHUNKS)
        def _():
            ebase = chunk * K_EDGES
            d_si = pltpu.async_copy(src_hbm.at[chunk], sidx, sem)
            d_di = pltpu.async_copy(dst_hbm.at[chunk], didx, sem)
            d_si.wait()
            d_di.wait()
            gathers = []
            for j in range(NSUB):
                gathers.append(pltpu.async_copy(
                    A_hbm.at[sidx.at[j]], ab.at[pl.ds(j * SUB, SUB)], sem))
                gathers.append(pltpu.async_copy(
                    B_hbm.at[didx.at[j]], bb.at[pl.ds(j * SUB, SUB)], sem))
            for g in gathers:
                g.wait()

            def _edge(e, _):
                a0 = ab[e, pl.ds(0, LANES)]
                a1 = ab[e, pl.ds(LANES, LANES)]
                b0 = bb[e, pl.ds(0, LANES)]
                b1 = bb[e, pl.ds(LANES, LANES)]
                ab[e, pl.ds(0, LANES)] = a0 + b0
                ab[e, pl.ds(LANES, LANES)] = a1 + b1
                return _

            lax.fori_loop(0, K_EDGES, _edge, None)
            pltpu.sync_copy(ab, comb_hbm.at[pl.ds(ebase, K_EDGES)])
        return _

    lax.fori_loop(0, T_STEPS, _chunk, None)
    plsc.subcore_barrier()


def _sc_comb_pass(a4, d4, src2, dst2):
    A_tab = a4.reshape(N, H)
    B_tab = d4.reshape(N, H)
    mesh = plsc.VectorSubcoreMesh(core_axis_name="c", subcore_axis_name="s")
    comb = pl.kernel(
        _sc_comb_body,
        out_type=jax.ShapeDtypeStruct((E, H), jnp.float32),
        mesh=mesh,
        compiler_params=pltpu.CompilerParams(use_tc_tiling_on_sc=False),
        scratch_types=[
            pltpu.VMEM((NSUB, SUB), jnp.int32),
            pltpu.VMEM((NSUB, SUB), jnp.int32),
            pltpu.VMEM((K_EDGES, H), jnp.float32),
            pltpu.VMEM((K_EDGES, H), jnp.float32),
            pltpu.SemaphoreType.DMA,
        ],
    )(A_tab, B_tab, src2, dst2)
    return comb.reshape(E // 4, 128)


def _sc_gine_body(C_hbm, ef_hbm, msg_hbm, src_hbm, dst_hbm,
                  efo_hbm, agg_hbm,
                  sidx, didx, efb, mb, cb, aggsh, sem):
    c = lax.axis_index("c")
    s = lax.axis_index("s")
    w = s * SC_CORES + c

    zero = jnp.zeros((LANES,), jnp.float32)

    def _zrow(e, _):
        cb[e, pl.ds(0, LANES)] = zero
        cb[e, pl.ds(LANES, LANES)] = zero
        return _

    lax.fori_loop(0, K_EDGES, _zrow, None)
    base = s * ROWS_PER_TILE

    def _zero_rows(nrows):
        off = 0
        while off < nrows:
            sz = min(K_EDGES, nrows - off)
            pltpu.sync_copy(cb.at[pl.ds(0, sz)],
                            aggsh.at[pl.ds(base + off, sz)])
            off += sz

    @pl.when(s < SC_SUBCORES - 1)
    def _():
        _zero_rows(ROWS_PER_TILE)

    @pl.when(s == SC_SUBCORES - 1)
    def _():
        _zero_rows(ROWS_LAST)

    plsc.subcore_barrier()

    def _chunk(t, _):
        chunk = w + t * SC_WORKERS

        @pl.when(chunk < NCHUNKS)
        def _():
            ebase = chunk * K_EDGES
            d_si = pltpu.async_copy(src_hbm.at[chunk], sidx, sem)
            d_di = pltpu.async_copy(dst_hbm.at[chunk], didx, sem)
            d_ef = pltpu.async_copy(ef_hbm.at[pl.ds(ebase, K_EDGES)], efb, sem)
            d_mg = pltpu.async_copy(msg_hbm.at[pl.ds(ebase, K_EDGES)], mb, sem)
            d_si.wait()
            gathers = []
            for j in range(NSUB):
                gathers.append(pltpu.async_copy(
                    C_hbm.at[sidx.at[j]], cb.at[pl.ds(j * SUB, SUB)], sem))
            d_di.wait()
            d_ef.wait()
            d_mg.wait()
            for g in gathers:
                g.wait()

            def _edge(e, _):
                f0 = efb[e, pl.ds(0, LANES)]
                f1 = efb[e, pl.ds(LANES, LANES)]
                m0 = mb[e, pl.ds(0, LANES)]
                m1 = mb[e, pl.ds(LANES, LANES)]
                g0 = jnp.maximum(f0 + m0, 0.0)
                g1 = jnp.maximum(f1 + m1, 0.0)
                efb[e, pl.ds(0, LANES)] = g0
                efb[e, pl.ds(LANES, LANES)] = g1
                c0 = cb[e, pl.ds(0, LANES)]
                c1 = cb[e, pl.ds(LANES, LANES)]
                cb[e, pl.ds(0, LANES)] = jnp.maximum(c0 + g0, 0.0)
                cb[e, pl.ds(LANES, LANES)] = jnp.maximum(c1 + g1, 0.0)
                return _

            lax.fori_loop(0, K_EDGES, _edge, None)

            for j in range(NSUB):
                pltpu.sync_copy(cb.at[pl.ds(j * SUB, SUB)],
                                aggsh.at[didx.at[j]], add=True)
            pltpu.sync_copy(efb, efo_hbm.at[pl.ds(ebase, K_EDGES)])
        return _

    lax.fori_loop(0, T_STEPS, _chunk, None)
    plsc.subcore_barrier()

    @pl.when(s < SC_SUBCORES - 1)
    def _():
        pltpu.sync_copy(aggsh.at[pl.ds(base, ROWS_PER_TILE)],
                        agg_hbm.at[c, pl.ds(base, ROWS_PER_TILE)])

    @pl.when(s == SC_SUBCORES - 1)
    def _():
        pltpu.sync_copy(aggsh.at[pl.ds(base, ROWS_LAST)],
                        agg_hbm.at[c, pl.ds(base, ROWS_LAST)])


def _sc_gine_pass(nf4, ef4, msg4, src2, dst2):
    C_tab = nf4.reshape(N, H)
    ef = ef4.reshape(E, H)
    msg = msg4.reshape(E, H)
    mesh = plsc.VectorSubcoreMesh(core_axis_name="c", subcore_axis_name="s")
    ef_o, agg = pl.kernel(
        _sc_gine_body,
        out_type=(jax.ShapeDtypeStruct((E, H), jnp.float32),
                  jax.ShapeDtypeStruct((SC_CORES, N, H), jnp.float32)),
        mesh=mesh,
        compiler_params=pltpu.CompilerParams(use_tc_tiling_on_sc=False),
        scratch_types=[
            pltpu.VMEM((NSUB, SUB), jnp.int32),
            pltpu.VMEM((NSUB, SUB), jnp.int32),
            pltpu.VMEM((K_EDGES, H), jnp.float32),
            pltpu.VMEM((K_EDGES, H), jnp.float32),
            pltpu.VMEM((K_EDGES, H), jnp.float32),
            pltpu.VMEM_SHARED((N, H), jnp.float32),
            pltpu.SemaphoreType.DMA,
        ],
    )(C_tab, ef, msg, src2, dst2)
    return ef_o.reshape(E // 4, 128), agg.reshape(SC_CORES, N // 4, 128)


def kernel(node_logits, edge_logits, edge_index, W_proj, b_proj, W_e1, b_e1,
           W_e2, b_e2, W_ap, b_ap, W_v, b_v, W_o, b_o, W_c1, b_c1, W_c2,
           b_c2, W_nh, b_nh, W_eh, b_eh):
    src2 = edge_index[0].reshape(NCHUNKS, NSUB, SUB)
    dst2 = edge_index[1].reshape(NCHUNKS, NSUB, SUB)
    nl4 = node_logits.reshape(N // 4, 16)
    elT = edge_logits.T
    el16 = jnp.concatenate(
        [elT[f].reshape(E // 16, 16) for f in range(EC)], axis=1)

    WpK = _kron(W_proj, 4)
    bpK = _tileb(b_proj, 4)
    WsK = _kron(W_ap[:H], 4)
    WdK = _kron(W_ap[H:], 4)
    bcK = _tileb(b_ap, 4)
    WvK = _kron(W_v, 4)
    bvK = _tileb(b_v, 4)
    WoK = _kron(W_o, 4)
    boK = _tileb(b_o, 4)
    W1K = jnp.einsum('jk,fc->fjkc', jnp.eye(16, dtype=jnp.float32),
                     W_e1).reshape(64, 512)
    b1K = _tileb(b_e1, 16)
    W2K = _kron(W_e2, 16)
    b2K = _tileb(b_e2, 16)
    Wc1K = _kron(W_c1, 4)
    bc1K = _tileb(b_c1, 4)
    Wc2K = _kron(W_c2, 4)
    bc2K = _tileb(b_c2, 4)
    WnhK = _kron(W_nh, 4)
    bnhK = _tileb(b_nh, 4)
    eye16 = jnp.eye(16, dtype=jnp.float32)
    WehP = jnp.stack([
        jnp.einsum('kf,ag->akfg', W_eh, eye16[4 * m:4 * m + 4]).reshape(
            128, 64)
        for m in range(4)])
    behP = jnp.repeat(b_eh, 16).reshape(1, 64)

    nf4, a4, d4 = _prep_node(nl4, WpK, bpK, WsK, WdK, bcK)
    ef4 = _prep_edge(el16, W1K, b1K, W2K, b2K)

    for it in range(2):
        comb4 = _sc_comb_pass(a4, d4, src2, dst2)
        msg4 = _msg(comb4, WvK, bvK, WoK, boK)
        ef4, agg4 = _sc_gine_pass(nf4, ef4, msg4, src2, dst2)
        if it == 0:
            nf4, a4, d4 = _node_update(nf4, agg4, Wc1K, bc1K, Wc2K, bc2K,
                                       WsK, WdK, bcK)

    node_out = _node_final(nf4, agg4, Wc1K, bc1K, Wc2K, bc2K, WnhK, bnhK)
    out16 = _edge_head(ef4, WehP, behP)
    edge_out = jnp.stack(
        [out16[:, 16 * f:16 * (f + 1)].reshape(E) for f in range(EC)],
        axis=1)
    return (node_out.reshape(N, NC), edge_out)


# speedup vs baseline: 1.1494x; 1.1494x over previous
import jax
import jax.numpy as jnp
from jax import lax
from jax.experimental import pallas as pl
from jax.experimental.pallas import tpu as pltpu
from jax.experimental.pallas import tpu_sc as plsc

N = 50000
E = 800000
H = 32
NC = 4
EC = 4

SC_CORES = 2
SC_SUBCORES = 16
SC_WORKERS = SC_CORES * SC_SUBCORES
LANES = 16

KC = 1000
SUBC = 100
NSUBC = KC // SUBC
NCHUNKS_C = E // KC
TC_STEPS = -(-NCHUNKS_C // SC_WORKERS)
KG = 250
SUBG = 125
NSUBG = KG // SUBG
NCHUNKS_G = E // KG
TG_STEPS = -(-NCHUNKS_G // SC_WORKERS)
ROWS_PER_TILE = 3128
ROWS_LAST = N - (SC_SUBCORES - 1) * ROWS_PER_TILE

NB = 4
NRB = N // 4 // NB


def _dot(x, w):
    return jnp.dot(x.astype(jnp.bfloat16), w.astype(jnp.bfloat16),
                   preferred_element_type=jnp.float32)


def _kron(w, p):
    return jnp.kron(jnp.eye(p, dtype=jnp.float32), w)


def _tileb(b, p):
    return jnp.tile(b.reshape(-1), p).reshape(1, -1)


def _prep_node_body(nl4, WpK, bpK, WsK, WdK, bcK, nf4, a4, d4):
    nf = _dot(nl4[0], WpK[...]) + bpK[...]
    nf4[0] = nf
    a4[0] = _dot(nf, WsK[...])
    d4[0] = _dot(nf, WdK[...]) + bcK[...]


def _prep_node(nl4, WpK, bpK, WsK, WdK, bcK):
    full = lambda s: pl.BlockSpec(s, lambda i: (0,) * len(s))
    blk = lambda m: pl.BlockSpec((1, NRB, m), lambda i: (i, 0, 0))
    o = jax.ShapeDtypeStruct((NB, NRB, 128), jnp.float32)
    return pl.pallas_call(
        _prep_node_body,
        grid=(NB,),
        out_shape=(o, o, o),
        in_specs=[blk(16), full((16, 128)), full((1, 128)),
                  full((128, 128)), full((128, 128)), full((1, 128))],
        out_specs=(blk(128), blk(128), blk(128)),
    )(nl4.reshape(NB, NRB, 16), WpK, bpK, WsK, WdK, bcK)


def _prep_edge_body(el16, W1K, b1K, W2K, b2K, ef4):
    t = jnp.maximum(_dot(el16[...], W1K[...]) + b1K[...], 0.0)
    e = _dot(t, W2K[...]) + b2K[...]
    ef4[...] = e.reshape(ef4.shape)


def _prep_edge(el16, W1K, b1K, W2K, b2K):
    RE = 2000
    g = (E // 16) // RE
    full = lambda s: pl.BlockSpec(s, lambda i: (0,) * len(s))
    return pl.pallas_call(
        _prep_edge_body,
        grid=(g,),
        out_shape=jax.ShapeDtypeStruct((E // 4, 128), jnp.float32),
        in_specs=[pl.BlockSpec((RE, 64), lambda i: (i, 0)),
                  full((64, 512)), full((1, 512)), full((512, 512)),
                  full((1, 512))],
        out_specs=pl.BlockSpec((4 * RE, 128), lambda i: (i, 0)),
    )(el16, W1K, b1K, W2K, b2K)


def _msg_body(comb4, WvK, bvK, WoK, boK, msg4):
    t = _dot(comb4[...], WvK[...]) + bvK[...]
    msg4[...] = _dot(t, WoK[...]) + boK[...]


def _msg(comb4, WvK, bvK, WoK, boK):
    RE = 8000
    g = (E // 4) // RE
    full = lambda s: pl.BlockSpec(s, lambda i: (0,) * len(s))
    return pl.pallas_call(
        _msg_body,
        grid=(g,),
        out_shape=jax.ShapeDtypeStruct((E // 4, 128), jnp.float32),
        in_specs=[pl.BlockSpec((RE, 128), lambda i: (i, 0)),
                  full((128, 128)), full((1, 128)), full((128, 128)),
                  full((1, 128))],
        out_specs=pl.BlockSpec((RE, 128), lambda i: (i, 0)),
    )(comb4, WvK, bvK, WoK, boK)


def _node_update_body(nf4, agg0, agg1, Wc1K, bc1K, Wc2K, bc2K, WsK, WdK, bcK,
                      nf4o, a4, d4):
    h = nf4[0] + agg0[0] + agg1[0]
    t = jnp.maximum(_dot(h, Wc1K[...]) + bc1K[...], 0.0)
    nf2 = jnp.maximum(_dot(t, Wc2K[...]) + bc2K[...], 0.0)
    nf4o[0] = nf2
    a4[0] = _dot(nf2, WsK[...])
    d4[0] = _dot(nf2, WdK[...]) + bcK[...]


def _node_update(nf4, agg4, Wc1K, bc1K, Wc2K, bc2K, WsK, WdK, bcK):
    full = lambda s: pl.BlockSpec(s, lambda i: (0,) * len(s))
    blk = lambda: pl.BlockSpec((1, NRB, 128), lambda i: (i, 0, 0))
    o = jax.ShapeDtypeStruct((NB, NRB, 128), jnp.float32)
    r3 = lambda x: x.reshape(NB, NRB, 128)
    return pl.pallas_call(
        _node_update_body,
        grid=(NB,),
        out_shape=(o, o, o),
        in_specs=[blk(), blk(), blk(),
                  full((128, 128)), full((1, 128)), full((128, 128)),
                  full((1, 128)), full((128, 128)), full((128, 128)),
                  full((1, 128))],
        out_specs=(blk(), blk(), blk()),
    )(r3(nf4), r3(agg4[0]), r3(agg4[1]), Wc1K, bc1K, Wc2K, bc2K, WsK, WdK,
      bcK)


def _node_final_body(nf4, agg0, agg1, Wc1K, bc1K, Wc2K, bc2K, WnhK, bnhK,
                     out):
    h = nf4[0] + agg0[0] + agg1[0]
    t = jnp.maximum(_dot(h, Wc1K[...]) + bc1K[...], 0.0)
    nf2 = jnp.maximum(_dot(t, Wc2K[...]) + bc2K[...], 0.0)
    out[0] = _dot(nf2, WnhK[...]) + bnhK[...]


def _node_final(nf4, agg4, Wc1K, bc1K, Wc2K, bc2K, WnhK, bnhK):
    full = lambda s: pl.BlockSpec(s, lambda i: (0,) * len(s))
    blk = lambda m: pl.BlockSpec((1, NRB, m), lambda i: (i, 0, 0))
    r3 = lambda x: x.reshape(NB, NRB, 128)
    return pl.pallas_call(
        _node_final_body,
        grid=(NB,),
        out_shape=jax.ShapeDtypeStruct((NB, NRB, 16), jnp.float32),
        in_specs=[blk(128), blk(128), blk(128),
                  full((128, 128)), full((1, 128)), full((128, 128)),
                  full((1, 128)), full((128, 16)), full((1, 16))],
        out_specs=blk(16),
    )(r3(nf4), r3(agg4[0]), r3(agg4[1]), Wc1K, bc1K, Wc2K, bc2K, WnhK, bnhK)


def _edge_head_body(ef4, WehP, behP, out):
    x3 = ef4[...].reshape(ef4.shape[0] // 4, 4, 128)
    acc = behP[...]
    for m in range(4):
        acc = acc + _dot(x3[:, m, :], WehP[m])
    out[...] = acc


def _edge_head(ef4, WehP, behP):
    RE = 8000
    g = (E // 4) // RE
    full = lambda s: pl.BlockSpec(s, lambda i: (0,) * len(s))
    return pl.pallas_call(
        _edge_head_body,
        grid=(g,),
        out_shape=jax.ShapeDtypeStruct((E // 16, 64), jnp.float32),
        in_specs=[pl.BlockSpec((RE, 128), lambda i: (i, 0)),
                  full((4, 128, 64)), full((1, 64))],
        out_specs=pl.BlockSpec((RE // 4, 64), lambda i: (i, 0)),
    )(ef4, WehP, behP)


def _sc_comb_body(A_hbm, B_hbm, src_hbm, dst_hbm, comb_hbm,
                  sidx, didx, ab, bb, sem):
    c = lax.axis_index("c")
    s = lax.axis_index("s")
    w = s * SC_CORES + c

    def _chunk(t, _):
        chunk = w + t * SC_WORKERS

        @pl.when(chunk < NCHUNKS_C)
        def _():
            ebase = chunk * KC
            d_si = pltpu.async_copy(src_hbm.at[chunk], sidx, sem)
            d_di = pltpu.async_copy(dst_hbm.at[chunk], didx, sem)
            d_si.wait()
            d_di.wait()
            gathers = []
            for j in range(NSUBC):
                gathers.append(pltpu.async_copy(
                    A_hbm.at[sidx.at[j]], ab.at[pl.ds(j * SUBC, SUBC)], sem))
                gathers.append(pltpu.async_copy(
                    B_hbm.at[didx.at[j]], bb.at[pl.ds(j * SUBC, SUBC)], sem))
            for g in gathers:
                g.wait()

            def _edge(e, _):
                a0 = ab[e, pl.ds(0, LANES)]
                a1 = ab[e, pl.ds(LANES, LANES)]
                b0 = bb[e, pl.ds(0, LANES)]
                b1 = bb[e, pl.ds(LANES, LANES)]
                ab[e, pl.ds(0, LANES)] = a0 + b0
                ab[e, pl.ds(LANES, LANES)] = a1 + b1
                return _

            lax.fori_loop(0, KC, _edge, None)
            pltpu.sync_copy(ab, comb_hbm.at[pl.ds(ebase, KC)])
        return _

    lax.fori_loop(0, TC_STEPS, _chunk, None)
    plsc.subcore_barrier()


def _sc_comb_pass(a4, d4, src2, dst2):
    A_tab = a4.reshape(N, H)
    B_tab = d4.reshape(N, H)
    mesh = plsc.VectorSubcoreMesh(core_axis_name="c", subcore_axis_name="s")
    comb = pl.kernel(
        _sc_comb_body,
        out_type=jax.ShapeDtypeStruct((E, H), jnp.float32),
        mesh=mesh,
        compiler_params=pltpu.CompilerParams(use_tc_tiling_on_sc=False),
        scratch_types=[
            pltpu.VMEM((NSUBC, SUBC), jnp.int32),
            pltpu.VMEM((NSUBC, SUBC), jnp.int32),
            pltpu.VMEM((KC, H), jnp.float32),
            pltpu.VMEM((KC, H), jnp.float32),
            pltpu.SemaphoreType.DMA,
        ],
    )(A_tab, B_tab, src2, dst2)
    return comb.reshape(E // 4, 128)


def _sc_gine_body(C_hbm, ef_hbm, msg_hbm, src_hbm, dst_hbm,
                  efo_hbm, agg_hbm,
                  sidx, didx, efb, mb, cb, aggsh, sem):
    c = lax.axis_index("c")
    s = lax.axis_index("s")
    w = s * SC_CORES + c

    zero = jnp.zeros((LANES,), jnp.float32)

    def _zrow(e, _):
        cb[e, pl.ds(0, LANES)] = zero
        cb[e, pl.ds(LANES, LANES)] = zero
        return _

    lax.fori_loop(0, KG, _zrow, None)
    base = s * ROWS_PER_TILE

    def _zero_rows(nrows):
        off = 0
        while off < nrows:
            sz = min(KG, nrows - off)
            pltpu.sync_copy(cb.at[pl.ds(0, sz)],
                            aggsh.at[pl.ds(base + off, sz)])
            off += sz

    @pl.when(s < SC_SUBCORES - 1)
    def _():
        _zero_rows(ROWS_PER_TILE)

    @pl.when(s == SC_SUBCORES - 1)
    def _():
        _zero_rows(ROWS_LAST)

    plsc.subcore_barrier()

    def _chunk(t, _):
        chunk = w + t * SC_WORKERS

        @pl.when(chunk < NCHUNKS_G)
        def _():
            ebase = chunk * KG
            d_si = pltpu.async_copy(src_hbm.at[chunk], sidx, sem)
            d_di = pltpu.async_copy(dst_hbm.at[chunk], didx, sem)
            d_ef = pltpu.async_copy(ef_hbm.at[pl.ds(ebase, KG)], efb, sem)
            d_mg = pltpu.async_copy(msg_hbm.at[pl.ds(ebase, KG)], mb, sem)
            d_si.wait()
            gathers = []
            for j in range(NSUBG):
                gathers.append(pltpu.async_copy(
                    C_hbm.at[sidx.at[j]], cb.at[pl.ds(j * SUBG, SUBG)], sem))
            d_di.wait()
            d_ef.wait()
            d_mg.wait()
            for g in gathers:
                g.wait()

            def _edge(e, _):
                f0 = efb[e, pl.ds(0, LANES)]
                f1 = efb[e, pl.ds(LANES, LANES)]
                m0 = mb[e, pl.ds(0, LANES)]
                m1 = mb[e, pl.ds(LANES, LANES)]
                g0 = jnp.maximum(f0 + m0, 0.0)
                g1 = jnp.maximum(f1 + m1, 0.0)
                efb[e, pl.ds(0, LANES)] = g0
                efb[e, pl.ds(LANES, LANES)] = g1
                c0 = cb[e, pl.ds(0, LANES)]
                c1 = cb[e, pl.ds(LANES, LANES)]
                cb[e, pl.ds(0, LANES)] = jnp.maximum(c0 + g0, 0.0)
                cb[e, pl.ds(LANES, LANES)] = jnp.maximum(c1 + g1, 0.0)
                return _

            lax.fori_loop(0, KG, _edge, None)

            for j in range(NSUBG):
                pltpu.sync_copy(cb.at[pl.ds(j * SUBG, SUBG)],
                                aggsh.at[didx.at[j]], add=True)
            pltpu.sync_copy(efb, efo_hbm.at[pl.ds(ebase, KG)])
        return _

    lax.fori_loop(0, TG_STEPS, _chunk, None)
    plsc.subcore_barrier()

    @pl.when(s < SC_SUBCORES - 1)
    def _():
        pltpu.sync_copy(aggsh.at[pl.ds(base, ROWS_PER_TILE)],
                        agg_hbm.at[c, pl.ds(base, ROWS_PER_TILE)])

    @pl.when(s == SC_SUBCORES - 1)
    def _():
        pltpu.sync_copy(aggsh.at[pl.ds(base, ROWS_LAST)],
                        agg_hbm.at[c, pl.ds(base, ROWS_LAST)])


def _sc_gine_pass(nf4, ef4, msg4, src2g, dst2g):
    C_tab = nf4.reshape(N, H)
    ef = ef4.reshape(E, H)
    msg = msg4.reshape(E, H)
    mesh = plsc.VectorSubcoreMesh(core_axis_name="c", subcore_axis_name="s")
    ef_o, agg = pl.kernel(
        _sc_gine_body,
        out_type=(jax.ShapeDtypeStruct((E, H), jnp.float32),
                  jax.ShapeDtypeStruct((SC_CORES, N, H), jnp.float32)),
        mesh=mesh,
        compiler_params=pltpu.CompilerParams(use_tc_tiling_on_sc=False),
        scratch_types=[
            pltpu.VMEM((NSUBG, SUBG), jnp.int32),
            pltpu.VMEM((NSUBG, SUBG), jnp.int32),
            pltpu.VMEM((KG, H), jnp.float32),
            pltpu.VMEM((KG, H), jnp.float32),
            pltpu.VMEM((KG, H), jnp.float32),
            pltpu.VMEM_SHARED((N, H), jnp.float32),
            pltpu.SemaphoreType.DMA,
        ],
    )(C_tab, ef, msg, src2g, dst2g)
    return ef_o.reshape(E // 4, 128), agg.reshape(SC_CORES, N // 4, 128)


def kernel(node_logits, edge_logits, edge_index, W_proj, b_proj, W_e1, b_e1,
           W_e2, b_e2, W_ap, b_ap, W_v, b_v, W_o, b_o, W_c1, b_c1, W_c2,
           b_c2, W_nh, b_nh, W_eh, b_eh):
    src2c = edge_index[0].reshape(NCHUNKS_C, NSUBC, SUBC)
    dst2c = edge_index[1].reshape(NCHUNKS_C, NSUBC, SUBC)
    src2g = edge_index[0].reshape(NCHUNKS_G, NSUBG, SUBG)
    dst2g = edge_index[1].reshape(NCHUNKS_G, NSUBG, SUBG)
    nl4 = node_logits.reshape(N // 4, 16)
    elT = edge_logits.T
    el16 = jnp.concatenate(
        [elT[f].reshape(E // 16, 16) for f in range(EC)], axis=1)

    WpK = _kron(W_proj, 4)
    bpK = _tileb(b_proj, 4)
    WsK = _kron(W_ap[:H], 4)
    WdK = _kron(W_ap[H:], 4)
    bcK = _tileb(b_ap, 4)
    WvK = _kron(W_v, 4)
    bvK = _tileb(b_v, 4)
    WoK = _kron(W_o, 4)
    boK = _tileb(b_o, 4)
    W1K = jnp.einsum('jk,fc->fjkc', jnp.eye(16, dtype=jnp.float32),
                     W_e1).reshape(64, 512)
    b1K = _tileb(b_e1, 16)
    W2K = _kron(W_e2, 16)
    b2K = _tileb(b_e2, 16)
    Wc1K = _kron(W_c1, 4)
    bc1K = _tileb(b_c1, 4)
    Wc2K = _kron(W_c2, 4)
    bc2K = _tileb(b_c2, 4)
    WnhK = _kron(W_nh, 4)
    bnhK = _tileb(b_nh, 4)
    eye16 = jnp.eye(16, dtype=jnp.float32)
    WehP = jnp.stack([
        jnp.einsum('kf,ag->akfg', W_eh, eye16[4 * m:4 * m + 4]).reshape(
            128, 64)
        for m in range(4)])
    behP = jnp.repeat(b_eh, 16).reshape(1, 64)

    nf4, a4, d4 = _prep_node(nl4, WpK, bpK, WsK, WdK, bcK)
    ef4 = _prep_edge(el16, W1K, b1K, W2K, b2K)

    for it in range(2):
        comb4 = _sc_comb_pass(a4, d4, src2c, dst2c)
        msg4 = _msg(comb4, WvK, bvK, WoK, boK)
        ef4, agg4 = _sc_gine_pass(nf4, ef4, msg4, src2g, dst2g)
        if it == 0:
            nf4, a4, d4 = _node_update(nf4, agg4, Wc1K, bc1K, Wc2K, bc2K,
                                       WsK, WdK, bcK)

    node_out = _node_final(nf4, agg4, Wc1K, bc1K, Wc2K, bc2K, WnhK, bnhK)
    out16 = _edge_head(ef4, WehP, behP)
    edge_out = jnp.stack(
        [out16[:, 16 * f:16 * (f + 1)].reshape(E) for f in range(EC)],
        axis=1)
    return (node_out.reshape(N, NC), edge_out)


# speedup vs baseline: 1.1551x; 1.0049x over previous
import jax
import jax.numpy as jnp
from jax import lax
from jax.experimental import pallas as pl
from jax.experimental.pallas import tpu as pltpu
from jax.experimental.pallas import tpu_sc as plsc

N = 50000
E = 800000
H = 32
NC = 4
EC = 4

SC_CORES = 2
SC_SUBCORES = 16
SC_WORKERS = SC_CORES * SC_SUBCORES
LANES = 16

KC = 1600
SUBC = 100
NSUBC = KC // SUBC
NCHUNKS_C = E // KC
TC_STEPS = -(-NCHUNKS_C // SC_WORKERS)
KG = 250
SUBG = 125
NSUBG = KG // SUBG
NCHUNKS_G = E // KG
TG_STEPS = -(-NCHUNKS_G // SC_WORKERS)
ROWS_PER_TILE = 3128
ROWS_LAST = N - (SC_SUBCORES - 1) * ROWS_PER_TILE

NB = 4
NRB = N // 4 // NB


def _dot(x, w):
    return jnp.dot(x.astype(jnp.bfloat16), w.astype(jnp.bfloat16),
                   preferred_element_type=jnp.float32)


def _kron(w, p):
    return jnp.kron(jnp.eye(p, dtype=jnp.float32), w)


def _tileb(b, p):
    return jnp.tile(b.reshape(-1), p).reshape(1, -1)


def _prep_node_body(nl4, WpK, bpK, WsK, WdK, bcK, nf4, a4, d4):
    nf = _dot(nl4[0], WpK[...]) + bpK[...]
    nf4[0] = nf
    a4[0] = _dot(nf, WsK[...])
    d4[0] = _dot(nf, WdK[...]) + bcK[...]


def _prep_node(nl4, WpK, bpK, WsK, WdK, bcK):
    full = lambda s: pl.BlockSpec(s, lambda i: (0,) * len(s))
    blk = lambda m: pl.BlockSpec((1, NRB, m), lambda i: (i, 0, 0))
    o = jax.ShapeDtypeStruct((NB, NRB, 128), jnp.float32)
    return pl.pallas_call(
        _prep_node_body,
        grid=(NB,),
        out_shape=(o, o, o),
        in_specs=[blk(16), full((16, 128)), full((1, 128)),
                  full((128, 128)), full((128, 128)), full((1, 128))],
        out_specs=(blk(128), blk(128), blk(128)),
    )(nl4.reshape(NB, NRB, 16), WpK, bpK, WsK, WdK, bcK)


def _prep_edge_body(el16, W1K, b1K, W2K, b2K, ef4):
    t = jnp.maximum(_dot(el16[...], W1K[...]) + b1K[...], 0.0)
    e = _dot(t, W2K[...]) + b2K[...]
    ef4[...] = e.reshape(ef4.shape)


def _prep_edge(el16, W1K, b1K, W2K, b2K):
    RE = 2000
    g = (E // 16) // RE
    full = lambda s: pl.BlockSpec(s, lambda i: (0,) * len(s))
    return pl.pallas_call(
        _prep_edge_body,
        grid=(g,),
        out_shape=jax.ShapeDtypeStruct((E // 4, 128), jnp.float32),
        in_specs=[pl.BlockSpec((RE, 64), lambda i: (i, 0)),
                  full((64, 512)), full((1, 512)), full((512, 512)),
                  full((1, 512))],
        out_specs=pl.BlockSpec((4 * RE, 128), lambda i: (i, 0)),
    )(el16, W1K, b1K, W2K, b2K)


def _msg_body(comb4, WvK, bvK, WoK, boK, msg4):
    t = _dot(comb4[...], WvK[...]) + bvK[...]
    msg4[...] = _dot(t, WoK[...]) + boK[...]


def _msg(comb4, WvK, bvK, WoK, boK):
    RE = 8000
    g = (E // 4) // RE
    full = lambda s: pl.BlockSpec(s, lambda i: (0,) * len(s))
    return pl.pallas_call(
        _msg_body,
        grid=(g,),
        out_shape=jax.ShapeDtypeStruct((E // 4, 128), jnp.float32),
        in_specs=[pl.BlockSpec((RE, 128), lambda i: (i, 0)),
                  full((128, 128)), full((1, 128)), full((128, 128)),
                  full((1, 128))],
        out_specs=pl.BlockSpec((RE, 128), lambda i: (i, 0)),
    )(comb4, WvK, bvK, WoK, boK)


def _node_update_body(nf4, agg0, agg1, Wc1K, bc1K, Wc2K, bc2K, WsK, WdK, bcK,
                      nf4o, a4, d4):
    h = nf4[0] + agg0[0] + agg1[0]
    t = jnp.maximum(_dot(h, Wc1K[...]) + bc1K[...], 0.0)
    nf2 = jnp.maximum(_dot(t, Wc2K[...]) + bc2K[...], 0.0)
    nf4o[0] = nf2
    a4[0] = _dot(nf2, WsK[...])
    d4[0] = _dot(nf2, WdK[...]) + bcK[...]


def _node_update(nf4, agg4, Wc1K, bc1K, Wc2K, bc2K, WsK, WdK, bcK):
    full = lambda s: pl.BlockSpec(s, lambda i: (0,) * len(s))
    blk = lambda: pl.BlockSpec((1, NRB, 128), lambda i: (i, 0, 0))
    o = jax.ShapeDtypeStruct((NB, NRB, 128), jnp.float32)
    r3 = lambda x: x.reshape(NB, NRB, 128)
    return pl.pallas_call(
        _node_update_body,
        grid=(NB,),
        out_shape=(o, o, o),
        in_specs=[blk(), blk(), blk(),
                  full((128, 128)), full((1, 128)), full((128, 128)),
                  full((1, 128)), full((128, 128)), full((128, 128)),
                  full((1, 128))],
        out_specs=(blk(), blk(), blk()),
    )(r3(nf4), r3(agg4[0]), r3(agg4[1]), Wc1K, bc1K, Wc2K, bc2K, WsK, WdK,
      bcK)


def _node_final_body(nf4, agg0, agg1, Wc1K, bc1K, Wc2K, bc2K, WnhK, bnhK,
                     out):
    h = nf4[0] + agg0[0] + agg1[0]
    t = jnp.maximum(_dot(h, Wc1K[...]) + bc1K[...], 0.0)
    nf2 = jnp.maximum(_dot(t, Wc2K[...]) + bc2K[...], 0.0)
    out[0] = _dot(nf2, WnhK[...]) + bnhK[...]


def _node_final(nf4, agg4, Wc1K, bc1K, Wc2K, bc2K, WnhK, bnhK):
    full = lambda s: pl.BlockSpec(s, lambda i: (0,) * len(s))
    blk = lambda m: pl.BlockSpec((1, NRB, m), lambda i: (i, 0, 0))
    r3 = lambda x: x.reshape(NB, NRB, 128)
    return pl.pallas_call(
        _node_final_body,
        grid=(NB,),
        out_shape=jax.ShapeDtypeStruct((NB, NRB, 16), jnp.float32),
        in_specs=[blk(128), blk(128), blk(128),
                  full((128, 128)), full((1, 128)), full((128, 128)),
                  full((1, 128)), full((128, 16)), full((1, 16))],
        out_specs=blk(16),
    )(r3(nf4), r3(agg4[0]), r3(agg4[1]), Wc1K, bc1K, Wc2K, bc2K, WnhK, bnhK)


def _edge_head_body(ef4, WehP, behP, out):
    x3 = ef4[...].reshape(ef4.shape[0] // 4, 4, 128)
    acc = behP[...]
    for m in range(4):
        acc = acc + _dot(x3[:, m, :], WehP[m])
    out[...] = acc


def _edge_head(ef4, WehP, behP):
    RE = 8000
    g = (E // 4) // RE
    full = lambda s: pl.BlockSpec(s, lambda i: (0,) * len(s))
    return pl.pallas_call(
        _edge_head_body,
        grid=(g,),
        out_shape=jax.ShapeDtypeStruct((E // 16, 64), jnp.float32),
        in_specs=[pl.BlockSpec((RE, 128), lambda i: (i, 0)),
                  full((4, 128, 64)), full((1, 64))],
        out_specs=pl.BlockSpec((RE // 4, 64), lambda i: (i, 0)),
    )(ef4, WehP, behP)


def _sc_comb_body(A_hbm, B_hbm, src_hbm, dst_hbm, comb_hbm,
                  sidx, didx, ab, bb, sem):
    c = lax.axis_index("c")
    s = lax.axis_index("s")
    w = s * SC_CORES + c

    def _chunk(t, _):
        chunk = w + t * SC_WORKERS

        @pl.when(chunk < NCHUNKS_C)
        def _():
            ebase = chunk * KC
            d_si = pltpu.async_copy(src_hbm.at[chunk], sidx, sem)
            d_di = pltpu.async_copy(dst_hbm.at[chunk], didx, sem)
            d_si.wait()
            d_di.wait()
            gathers = []
            for j in range(NSUBC):
                gathers.append(pltpu.async_copy(
                    A_hbm.at[sidx.at[j]], ab.at[pl.ds(j * SUBC, SUBC)], sem))
                gathers.append(pltpu.async_copy(
                    B_hbm.at[didx.at[j]], bb.at[pl.ds(j * SUBC, SUBC)], sem))
            for g in gathers:
                g.wait()

            def _edge(e, _):
                a0 = ab[e, pl.ds(0, LANES)]
                a1 = ab[e, pl.ds(LANES, LANES)]
                b0 = bb[e, pl.ds(0, LANES)]
                b1 = bb[e, pl.ds(LANES, LANES)]
                ab[e, pl.ds(0, LANES)] = a0 + b0
                ab[e, pl.ds(LANES, LANES)] = a1 + b1
                return _

            lax.fori_loop(0, KC, _edge, None)
            pltpu.sync_copy(ab, comb_hbm.at[pl.ds(ebase, KC)])
        return _

    lax.fori_loop(0, TC_STEPS, _chunk, None)
    plsc.subcore_barrier()


def _sc_comb_pass(a4, d4, src2, dst2):
    A_tab = a4.reshape(N, H)
    B_tab = d4.reshape(N, H)
    mesh = plsc.VectorSubcoreMesh(core_axis_name="c", subcore_axis_name="s")
    comb = pl.kernel(
        _sc_comb_body,
        out_type=jax.ShapeDtypeStruct((E, H), jnp.float32),
        mesh=mesh,
        compiler_params=pltpu.CompilerParams(use_tc_tiling_on_sc=False),
        scratch_types=[
            pltpu.VMEM((NSUBC, SUBC), jnp.int32),
            pltpu.VMEM((NSUBC, SUBC), jnp.int32),
            pltpu.VMEM((KC, H), jnp.float32),
            pltpu.VMEM((KC, H), jnp.float32),
            pltpu.SemaphoreType.DMA,
        ],
    )(A_tab, B_tab, src2, dst2)
    return comb.reshape(E // 4, 128)


def _sc_gine_body(C_hbm, ef_hbm, msg_hbm, src_hbm, dst_hbm,
                  efo_hbm, agg_hbm,
                  sidx, didx, efb, mb, cb, aggsh, sem):
    c = lax.axis_index("c")
    s = lax.axis_index("s")
    w = s * SC_CORES + c

    zero = jnp.zeros((LANES,), jnp.float32)

    def _zrow(e, _):
        cb[e, pl.ds(0, LANES)] = zero
        cb[e, pl.ds(LANES, LANES)] = zero
        return _

    lax.fori_loop(0, KG, _zrow, None)
    base = s * ROWS_PER_TILE

    def _zero_rows(nrows):
        off = 0
        while off < nrows:
            sz = min(KG, nrows - off)
            pltpu.sync_copy(cb.at[pl.ds(0, sz)],
                            aggsh.at[pl.ds(base + off, sz)])
            off += sz

    @pl.when(s < SC_SUBCORES - 1)
    def _():
        _zero_rows(ROWS_PER_TILE)

    @pl.when(s == SC_SUBCORES - 1)
    def _():
        _zero_rows(ROWS_LAST)

    plsc.subcore_barrier()

    def _chunk(t, _):
        chunk = w + t * SC_WORKERS

        @pl.when(chunk < NCHUNKS_G)
        def _():
            ebase = chunk * KG
            d_si = pltpu.async_copy(src_hbm.at[chunk], sidx, sem)
            d_di = pltpu.async_copy(dst_hbm.at[chunk], didx, sem)
            d_ef = pltpu.async_copy(ef_hbm.at[pl.ds(ebase, KG)], efb, sem)
            d_mg = pltpu.async_copy(msg_hbm.at[pl.ds(ebase, KG)], mb, sem)
            d_si.wait()
            gathers = []
            for j in range(NSUBG):
                gathers.append(pltpu.async_copy(
                    C_hbm.at[sidx.at[j]], cb.at[pl.ds(j * SUBG, SUBG)], sem))
            d_di.wait()
            d_ef.wait()
            d_mg.wait()
            for g in gathers:
                g.wait()

            def _edge(e, _):
                f0 = efb[e, pl.ds(0, LANES)]
                f1 = efb[e, pl.ds(LANES, LANES)]
                m0 = mb[e, pl.ds(0, LANES)]
                m1 = mb[e, pl.ds(LANES, LANES)]
                g0 = jnp.maximum(f0 + m0, 0.0)
                g1 = jnp.maximum(f1 + m1, 0.0)
                efb[e, pl.ds(0, LANES)] = g0
                efb[e, pl.ds(LANES, LANES)] = g1
                c0 = cb[e, pl.ds(0, LANES)]
                c1 = cb[e, pl.ds(LANES, LANES)]
                cb[e, pl.ds(0, LANES)] = jnp.maximum(c0 + g0, 0.0)
                cb[e, pl.ds(LANES, LANES)] = jnp.maximum(c1 + g1, 0.0)
                return _

            lax.fori_loop(0, KG, _edge, None)

            for j in range(NSUBG):
                pltpu.sync_copy(cb.at[pl.ds(j * SUBG, SUBG)],
                                aggsh.at[didx.at[j]], add=True)
            pltpu.sync_copy(efb, efo_hbm.at[pl.ds(ebase, KG)])
        return _

    lax.fori_loop(0, TG_STEPS, _chunk, None)
    plsc.subcore_barrier()

    @pl.when(s < SC_SUBCORES - 1)
    def _():
        pltpu.sync_copy(aggsh.at[pl.ds(base, ROWS_PER_TILE)],
                        agg_hbm.at[c, pl.ds(base, ROWS_PER_TILE)])

    @pl.when(s == SC_SUBCORES - 1)
    def _():
        pltpu.sync_copy(aggsh.at[pl.ds(base, ROWS_LAST)],
                        agg_hbm.at[c, pl.ds(base, ROWS_LAST)])


def _sc_gine_pass(nf4, ef4, msg4, src2g, dst2g):
    C_tab = nf4.reshape(N, H)
    ef = ef4.reshape(E, H)
    msg = msg4.reshape(E, H)
    mesh = plsc.VectorSubcoreMesh(core_axis_name="c", subcore_axis_name="s")
    ef_o, agg = pl.kernel(
        _sc_gine_body,
        out_type=(jax.ShapeDtypeStruct((E, H), jnp.float32),
                  jax.ShapeDtypeStruct((SC_CORES, N, H), jnp.float32)),
        mesh=mesh,
        compiler_params=pltpu.CompilerParams(use_tc_tiling_on_sc=False),
        scratch_types=[
            pltpu.VMEM((NSUBG, SUBG), jnp.int32),
            pltpu.VMEM((NSUBG, SUBG), jnp.int32),
            pltpu.VMEM((KG, H), jnp.float32),
            pltpu.VMEM((KG, H), jnp.float32),
            pltpu.VMEM((KG, H), jnp.float32),
            pltpu.VMEM_SHARED((N, H), jnp.float32),
            pltpu.SemaphoreType.DMA,
        ],
    )(C_tab, ef, msg, src2g, dst2g)
    return ef_o.reshape(E // 4, 128), agg.reshape(SC_CORES, N // 4, 128)


def kernel(node_logits, edge_logits, edge_index, W_proj, b_proj, W_e1, b_e1,
           W_e2, b_e2, W_ap, b_ap, W_v, b_v, W_o, b_o, W_c1, b_c1, W_c2,
           b_c2, W_nh, b_nh, W_eh, b_eh):
    src2c = edge_index[0].reshape(NCHUNKS_C, NSUBC, SUBC)
    dst2c = edge_index[1].reshape(NCHUNKS_C, NSUBC, SUBC)
    src2g = edge_index[0].reshape(NCHUNKS_G, NSUBG, SUBG)
    dst2g = edge_index[1].reshape(NCHUNKS_G, NSUBG, SUBG)
    nl4 = node_logits.reshape(N // 4, 16)
    elT = edge_logits.T
    el16 = jnp.concatenate(
        [elT[f].reshape(E // 16, 16) for f in range(EC)], axis=1)

    WpK = _kron(W_proj, 4)
    bpK = _tileb(b_proj, 4)
    WsK = _kron(W_ap[:H], 4)
    WdK = _kron(W_ap[H:], 4)
    bcK = _tileb(b_ap, 4)
    WvK = _kron(W_v, 4)
    bvK = _tileb(b_v, 4)
    WoK = _kron(W_o, 4)
    boK = _tileb(b_o, 4)
    W1K = jnp.einsum('jk,fc->fjkc', jnp.eye(16, dtype=jnp.float32),
                     W_e1).reshape(64, 512)
    b1K = _tileb(b_e1, 16)
    W2K = _kron(W_e2, 16)
    b2K = _tileb(b_e2, 16)
    Wc1K = _kron(W_c1, 4)
    bc1K = _tileb(b_c1, 4)
    Wc2K = _kron(W_c2, 4)
    bc2K = _tileb(b_c2, 4)
    WnhK = _kron(W_nh, 4)
    bnhK = _tileb(b_nh, 4)
    eye16 = jnp.eye(16, dtype=jnp.float32)
    WehP = jnp.stack([
        jnp.einsum('kf,ag->akfg', W_eh, eye16[4 * m:4 * m + 4]).reshape(
            128, 64)
        for m in range(4)])
    behP = jnp.repeat(b_eh, 16).reshape(1, 64)

    nf4, a4, d4 = _prep_node(nl4, WpK, bpK, WsK, WdK, bcK)
    ef4 = _prep_edge(el16, W1K, b1K, W2K, b2K)

    for it in range(2):
        comb4 = _sc_comb_pass(a4, d4, src2c, dst2c)
        msg4 = _msg(comb4, WvK, bvK, WoK, boK)
        ef4, agg4 = _sc_gine_pass(nf4, ef4, msg4, src2g, dst2g)
        if it == 0:
            nf4, a4, d4 = _node_update(nf4, agg4, Wc1K, bc1K, Wc2K, bc2K,
                                       WsK, WdK, bcK)

    node_out = _node_final(nf4, agg4, Wc1K, bc1K, Wc2K, bc2K, WnhK, bnhK)
    out16 = _edge_head(ef4, WehP, behP)
    edge_out = jnp.stack(
        [out16[:, 16 * f:16 * (f + 1)].reshape(E) for f in range(EC)],
        axis=1)
    return (node_out.reshape(N, NC), edge_out)
